# baseline (speedup 1.0000x reference)
import jax
import jax.numpy as jnp
from jax import lax
from jax.experimental import pallas as pl

_ROWS = 16384
_BITS = 8
_GRID = 16
_BLOCK_ROWS = _ROWS // _GRID


def _full_adder_bits(a, b, c):
    axb = a + b - 2.0 * a * b
    s = axb + c - 2.0 * axb * c
    ab = a * b
    cx = c * axb
    carry = ab + cx - ab * cx
    return s, carry


def _body(w1_ref, w2_ref, out_ref):
    b1 = (jnp.sign(w1_ref[...]) + 1.0) * 0.5
    b2 = (jnp.sign(w2_ref[...]) + 1.0) * 0.5

    col = lax.broadcasted_iota(jnp.int32, (1, _BITS), 1)
    c = jnp.zeros((1, 1), jnp.float32)
    row = jnp.zeros((1, _BITS), jnp.float32)
    for i in range(_BITS):
        s, c = _full_adder_bits(b1[:, i : i + 1], b2[:, i : i + 1], c)
        row = row + jnp.where(col == i, s * 2.0 - 1.0, 0.0)

    out_ref[...] = jnp.broadcast_to(row, (_BLOCK_ROWS, _BITS))


def kernel(input, W1, W2):
    del input
    return pl.pallas_call(
        _body,
        grid=(_GRID,),
        in_specs=[
            pl.BlockSpec((1, _BITS), lambda i: (0, 0)),
            pl.BlockSpec((1, _BITS), lambda i: (0, 0)),
        ],
        out_specs=pl.BlockSpec((_BLOCK_ROWS, _BITS), lambda i: (i, 0)),
        out_shape=jax.ShapeDtypeStruct((_ROWS, _BITS), jnp.float32),
    )(W1, W2)


# speedup vs baseline: 1.3454x; 1.3454x over previous
import jax
import jax.numpy as jnp
from jax import lax
from jax.experimental import pallas as pl
from jax.experimental.pallas import tpu as pltpu

_ROWS = 16384
_BITS = 8
_GRID = 16
_BLOCK_ROWS = _ROWS // _GRID


def _full_adder_bits(a, b, c):
    axb = a + b - 2.0 * a * b
    s = axb + c - 2.0 * axb * c
    ab = a * b
    cx = c * axb
    carry = ab + cx - ab * cx
    return s, carry


def _body(w1_ref, w2_ref, out_ref, row_ref):
    @pl.when(pl.program_id(0) == 0)
    def _compute_row():
        b1 = (jnp.sign(w1_ref[...]) + 1.0) * 0.5
        b2 = (jnp.sign(w2_ref[...]) + 1.0) * 0.5

        col = lax.broadcasted_iota(jnp.int32, (1, _BITS), 1)
        c = jnp.zeros((1, 1), jnp.float32)
        row = jnp.zeros((1, _BITS), jnp.float32)
        for i in range(_BITS):
            s, c = _full_adder_bits(b1[:, i : i + 1], b2[:, i : i + 1], c)
            row = row + jnp.where(col == i, s * 2.0 - 1.0, 0.0)
        row_ref[...] = row

    out_ref[...] = jnp.broadcast_to(row_ref[...], (_BLOCK_ROWS, _BITS))


def kernel(input, W1, W2):
    del input
    return pl.pallas_call(
        _body,
        grid=(_GRID,),
        in_specs=[
            pl.BlockSpec((1, _BITS), lambda i: (0, 0)),
            pl.BlockSpec((1, _BITS), lambda i: (0, 0)),
        ],
        out_specs=pl.BlockSpec((_BLOCK_ROWS, _BITS), lambda i: (i, 0)),
        out_shape=jax.ShapeDtypeStruct((_ROWS, _BITS), jnp.float32),
        scratch_shapes=[pltpu.VMEM((1, _BITS), jnp.float32)],
    )(W1, W2)


# speedup vs baseline: 8.6666x; 6.4417x over previous
import jax
import jax.numpy as jnp
from jax import lax
from jax.experimental import pallas as pl

_ROWS = 16384
_BITS = 8


def _full_adder_bits(a, b, c):
    axb = a + b - 2.0 * a * b
    s = axb + c - 2.0 * axb * c
    ab = a * b
    cx = c * axb
    carry = ab + cx - ab * cx
    return s, carry


def _body(w1_ref, w2_ref, out_ref):
    b1 = (jnp.sign(w1_ref[...]) + 1.0) * 0.5
    b2 = (jnp.sign(w2_ref[...]) + 1.0) * 0.5

    subl = lax.broadcasted_iota(jnp.int32, (_BITS, 1), 0)
    c = jnp.zeros((1, 1), jnp.float32)
    col = jnp.zeros((_BITS, 1), jnp.float32)
    for i in range(_BITS):
        s, c = _full_adder_bits(b1[:, i : i + 1], b2[:, i : i + 1], c)
        col = col + jnp.where(subl == i, s * 2.0 - 1.0, 0.0)

    out_ref[...] = jnp.broadcast_to(col, (_BITS, _ROWS))


def kernel(input, W1, W2):
    del input
    out_t = pl.pallas_call(
        _body,
        out_shape=jax.ShapeDtypeStruct((_BITS, _ROWS), jnp.float32),
    )(W1, W2)
    return out_t.T


# speedup vs baseline: 11.6477x; 1.3440x over previous
import jax
import jax.numpy as jnp
from jax import lax
from jax.experimental import pallas as pl
from jax.experimental.pallas import tpu as pltpu

_ROWS = 16384
_BITS = 8


def _full_adder_bits(a, b, c):
    axb = a + b - 2.0 * a * b
    s = axb + c - 2.0 * axb * c
    ab = a * b
    cx = c * axb
    carry = ab + cx - ab * cx
    return s, carry


def _body(w1_ref, w2_ref, out_ref):
    subl = lax.broadcasted_iota(jnp.int32, (_BITS, 1), 0)
    c = jnp.float32(0.0)
    col = jnp.zeros((_BITS, 1), jnp.float32)
    for i in range(_BITS):
        a = (jnp.sign(w1_ref[0, i]) + 1.0) * 0.5
        b = (jnp.sign(w2_ref[0, i]) + 1.0) * 0.5
        s, c = _full_adder_bits(a, b, c)
        col = jnp.where(subl == i, s * 2.0 - 1.0, col)

    out_ref[...] = jnp.broadcast_to(col, (_BITS, _ROWS))


def kernel(input, W1, W2):
    del input
    out_t = pl.pallas_call(
        _body,
        in_specs=[
            pl.BlockSpec(memory_space=pltpu.SMEM),
            pl.BlockSpec(memory_space=pltpu.SMEM),
        ],
        out_shape=jax.ShapeDtypeStruct((_BITS, _ROWS), jnp.float32),
    )(W1, W2)
    return out_t.T
